# baseline (speedup 1.0000x reference)
import functools

import jax
import jax.numpy as jnp
from jax import lax
from jax.experimental import pallas as pl
from jax.experimental.pallas import tpu as pltpu
from jax.experimental.pallas import tpu_sc as plsc

B = 16384
NDISC = 26
NCONT = 13
VOCAB = 100001
EDIM = 3
NROWS = EDIM * NDISC

LINE = 128
TILEW = 8 * LINE
RTF = VOCAB // LINE
TAIL = VOCAB - RTF * LINE
TTBLK = (RTF + 1) * TILEW
NOCT = NDISC // 8
NSIDE = NDISC - 8 * NOCT
SIDEBASE = EDIM * NOCT * TTBLK
VOCABP = 100008
IMG = SIDEBASE + EDIM * NSIDE * VOCABP
TAILSTRIDE = LINE
NTAILS = EDIM * 8 * NOCT

NC = 2
NS = 16
L = 16
NW = NC * NS
BPW = B // NW
CHUNK = 128
CPF = BPW // CHUNK
NKF = EDIM * CPF
NCHUNK = NDISC * NKF
NBUF = 4
SLOT = EDIM * BPW


def _wide_body(xt_hbm, tab_hbm, wb_hbm, out_hbm,
               xv, idxv, rows, acc, wbv,
               sem0, sem1, sem2, sem3):
  sems = (sem0, sem1, sem2, sem3)
  cid = lax.axis_index("c")
  sid = lax.axis_index("s")
  wid = sid * NC + cid
  base = wid * BPW

  pltpu.sync_copy(xt_hbm.at[:, pl.ds(base, BPW)], xv)
  pltpu.sync_copy(wb_hbm, wbv)

  for i in range(NDISC):
    row = 2 * i

    def vec_body(u, _, row=row, i=i):
      v = xv[row, pl.ds(u * L, L)]
      r = v.astype(jnp.int32)
      if i < 8 * NOCT:
        adr = ((r >> 7) << 10) + (r & 127)
      else:
        adr = r
      o = u // (CHUNK // L)
      s = (u % (CHUNK // L)) * L
      for d in range(EDIM):
        if i < 8 * NOCT:
          cdi = (d * NOCT + i // 8) * TTBLK + (i % 8) * LINE
        else:
          cdi = SIDEBASE + (d * NSIDE + (i - 8 * NOCT)) * VOCABP
        idxv[i * NKF + d * CPF + o, pl.ds(s, L)] = adr + cdi
      return 0

    lax.fori_loop(0, BPW // L, vec_body, 0)

  def fire(f):
    slot = f % NBUF
    for d in range(EDIM):
      for o in range(CPF):
        k = f * NKF + d * CPF + o
        pltpu.make_async_copy(
            tab_hbm.at[idxv.at[k]],
            rows.at[pl.ds(slot * SLOT + d * BPW + o * CHUNK, CHUNK)],
            sems[slot]).start()

  def drain(f):
    slot = f % NBUF
    for d in range(EDIM):
      for o in range(CPF):
        k = f * NKF + d * CPF + o
        pltpu.make_async_copy(
            tab_hbm.at[idxv.at[k]],
            rows.at[pl.ds(slot * SLOT + d * BPW + o * CHUNK, CHUNK)],
            sems[slot]).wait()

  for f in range(NBUF - 1):
    fire(f)

  wsp = [[wbv[d * NCONT + j, pl.ds(0, L)] for j in range(NCONT)]
         for d in range(EDIM)]
  bsp = [wbv[NCONT * EDIM + d, pl.ds(0, L)] for d in range(EDIM)]

  def cont_chunk(c, _):
    s = c * L
    a = [bsp[d] for d in range(EDIM)]
    for j in range(NCONT):
      v = xv[2 * (NDISC + j), pl.ds(s, L)]
      m = xv[2 * (NDISC + j) + 1, pl.ds(s, L)]
      u = plsc.bitcast(v * m, jnp.int32)
      u = (u + 0x7FFF + ((u >> 16) & 1)) & ~0xFFFF
      cv = plsc.bitcast(u, jnp.float32)
      for d in range(EDIM):
        a[d] = a[d] + cv * wsp[d][j]
    for d in range(EDIM):
      acc[d, pl.ds(s, L)] = a[d]
    return 0

  lax.fori_loop(0, BPW // L, cont_chunk, 0)

  for f in range(NDISC):
    slot = f % NBUF
    drain(f)
    mrow = 2 * f + 1
    rbase = slot * SLOT

    def emb_chunk(c, _, mrow=mrow, rbase=rbase):
      s = c * L
      m = xv[mrow, pl.ds(s, L)]
      for d in range(EDIM):
        g = rows[pl.ds(rbase + d * BPW + s, L)]
        plsc.addupdate(acc.at[d, pl.ds(s, L)], g * m)
      return 0

    lax.fori_loop(0, BPW // L, emb_chunk, 0)
    nxt = f + NBUF - 1
    if nxt < NDISC:
      fire(nxt)

  pltpu.sync_copy(acc, out_hbm.at[:, pl.ds(base, BPW)])


@functools.partial(
    pl.kernel,
    out_type=jax.ShapeDtypeStruct((EDIM, B), jnp.float32),
    mesh=plsc.VectorSubcoreMesh(core_axis_name="c", subcore_axis_name="s",
                                num_cores=NC, num_subcores=NS),
    compiler_params=pltpu.CompilerParams(needs_layout_passes=False,
                                         use_tc_tiling_on_sc=False),
    scratch_types=[
        pltpu.VMEM((2 * (NDISC + NCONT), BPW), jnp.float32),
        pltpu.VMEM((NCHUNK, CHUNK), jnp.int32),
        pltpu.VMEM((NBUF * SLOT,), jnp.float32),
        pltpu.VMEM((EDIM, BPW), jnp.float32),
        pltpu.VMEM((48, L), jnp.float32),
        pltpu.SemaphoreType.DMA,
        pltpu.SemaphoreType.DMA,
        pltpu.SemaphoreType.DMA,
        pltpu.SemaphoreType.DMA,
    ],
)
def _wide_sc(xt_hbm, tab_hbm, wb_hbm, out_hbm, xv, idxv, rows, acc, wbv,
             sem0, sem1, sem2, sem3):
  _wide_body(xt_hbm, tab_hbm, wb_hbm, out_hbm, xv, idxv, rows, acc, wbv,
             sem0, sem1, sem2, sem3)


NSLABT = EDIM * NOCT * RTF
TILES_PER_W = (NSLABT + NW - 1) // NW
SIDET = (EDIM * NSIDE * VOCABP + TILEW - 1) // TILEW
SIDET0 = EDIM * NOCT * (RTF + 1)
TILES = SIDET0 + SIDET
TPW = 6


def _detile_body(tab3_hbm, tail_hbm, side_hbm, out_hbm, sem):
  cid = lax.axis_index("c")
  sid = lax.axis_index("s")
  wid = sid * NC + cid
  lo = wid * TILES_PER_W
  hi = jnp.minimum(lo + TILES_PER_W, NSLABT)

  def tile_copy(u):
    blk = u // RTF
    rt = u % RTF
    d = blk // NOCT
    tt = blk % NOCT
    return pltpu.make_async_copy(
        tab3_hbm.at[d, pl.ds(pl.multiple_of(tt * 8, 8), 8),
                    pl.ds(pl.multiple_of(rt * LINE, LINE), LINE)],
        out_hbm.at[blk * (RTF + 1) + rt],
        sem)

  def fire_tile(u, _):
    tile_copy(u).start()
    return 0

  def wait_tile(u, _):
    tile_copy(u).wait()
    return 0

  extra = []
  for w in range(12):
    for q in range(TPW):
      rowid = w * TPW + q
      d, t = rowid // (8 * NOCT), rowid % (8 * NOCT)
      tailtile = (d * NOCT + t // 8) * (RTF + 1) + RTF
      extra.append((w, pltpu.make_async_copy(
          tail_hbm.at[pl.ds(rowid * TAILSTRIDE, TAILSTRIDE)],
          out_hbm.at[tailtile, t % 8, pl.ds(0, TAILSTRIDE)],
          sem)))
  for h in range(2):
    extra.append((30 + h, pltpu.make_async_copy(
        side_hbm.at[pl.ds(h * (SIDET // 2), SIDET // 2)],
        out_hbm.at[pl.ds(SIDET0 + h * (SIDET // 2), SIDET // 2)],
        sem)))

  lax.fori_loop(lo, hi, fire_tile, 0)
  for w, cp in extra:
    @pl.when(wid == w)
    def _(cp=cp):
      cp.start()
  lax.fori_loop(lo, hi, wait_tile, 0)
  for w, cp in extra:
    @pl.when(wid == w)
    def _(cp=cp):
      cp.wait()


@functools.partial(
    pl.kernel,
    out_type=jax.ShapeDtypeStruct((TILES, 8, LINE), jnp.float32),
    mesh=plsc.VectorSubcoreMesh(core_axis_name="c", subcore_axis_name="s",
                                num_cores=NC, num_subcores=NS),
    compiler_params=pltpu.CompilerParams(needs_layout_passes=False,
                                         use_tc_tiling_on_sc=True),
    scratch_types=[
        pltpu.SemaphoreType.DMA,
    ],
)
def _detile_sc(tab3_hbm, tail_hbm, side_hbm, out_hbm, sem):
  _detile_body(tab3_hbm, tail_hbm, side_hbm, out_hbm, sem)


def kernel(x, tables, W, b):
  xt = x.T
  tab3 = tables.transpose(2, 0, 1)
  tailp = jnp.pad(tab3[:, :8 * NOCT, RTF * LINE:],
                  ((0, 0), (0, 0), (0, TAILSTRIDE - TAIL))).reshape(-1)
  sidep = jnp.pad(tab3[:, 8 * NOCT:, :],
                  ((0, 0), (0, 0), (0, VOCABP - VOCAB))).reshape(-1)
  sidep = jnp.pad(sidep, (0, SIDET * TILEW - sidep.shape[0]))
  img = _detile_sc(tab3, tailp, sidep.reshape(SIDET, 8, LINE)).reshape(-1)
  wb = jnp.concatenate(
      [W.reshape(-1), b, jnp.zeros((48 - NCONT * EDIM - EDIM,), jnp.float32)])
  wb = jnp.broadcast_to(wb[:, None], (48, L))
  out_t = _wide_sc(xt, img, wb)
  return out_t.T


# speedup vs baseline: 2.3226x; 2.3226x over previous
import functools

import jax
import jax.numpy as jnp
from jax import lax
from jax.experimental import pallas as pl
from jax.experimental.pallas import tpu as pltpu
from jax.experimental.pallas import tpu_sc as plsc

B = 16384
NDISC = 26
NCONT = 13
VOCAB = 100001
EDIM = 3
PLANE = NDISC * VOCAB

NC = 2
NS = 16
L = 16
NW = NC * NS
BPW = B // NW
CHUNK = 128
CPF = BPW // CHUNK
NKF = EDIM * CPF
NCHUNK = NDISC * NKF
NBUF = 4
SLOT = EDIM * BPW


def _wide_body(xt_hbm, tab_hbm, wb_hbm, out_hbm,
               xv, idxv, rows, acc, wbv,
               sem0, sem1, sem2, sem3):
  sems = (sem0, sem1, sem2, sem3)
  cid = lax.axis_index("c")
  sid = lax.axis_index("s")
  wid = sid * NC + cid
  base = wid * BPW

  pltpu.sync_copy(xt_hbm.at[:, pl.ds(base, BPW)], xv)
  pltpu.sync_copy(wb_hbm, wbv)

  def idx_feature(i, _):
    row = 2 * i
    tbase = i * VOCAB

    def vec_body(u, _):
      v = xv[row, pl.ds(u * L, L)]
      r = v.astype(jnp.int32) + tbase
      o = u // (CHUNK // L)
      s = (u % (CHUNK // L)) * L
      for d in range(EDIM):
        idxv[i * NKF + d * CPF + o, pl.ds(s, L)] = r + d * PLANE
      return 0

    lax.fori_loop(0, BPW // L, vec_body, 0)
    return 0

  lax.fori_loop(0, NDISC, idx_feature, 0)

  def fire(f):
    slot = f % NBUF
    for d in range(EDIM):
      for o in range(CPF):
        k = f * NKF + d * CPF + o
        pltpu.make_async_copy(
            tab_hbm.at[idxv.at[k]],
            rows.at[pl.ds(slot * SLOT + d * BPW + o * CHUNK, CHUNK)],
            sems[slot]).start()

  def drain(f):
    slot = f % NBUF
    for d in range(EDIM):
      for o in range(CPF):
        k = f * NKF + d * CPF + o
        pltpu.make_async_copy(
            tab_hbm.at[idxv.at[k]],
            rows.at[pl.ds(slot * SLOT + d * BPW + o * CHUNK, CHUNK)],
            sems[slot]).wait()

  for f in range(NBUF - 1):
    fire(f)

  wsp = [[wbv[d * NCONT + j, pl.ds(0, L)] for j in range(NCONT)]
         for d in range(EDIM)]
  bsp = [wbv[NCONT * EDIM + d, pl.ds(0, L)] for d in range(EDIM)]

  def cont_chunk(c, _):
    s = c * L
    a = [bsp[d] for d in range(EDIM)]
    for j in range(NCONT):
      v = xv[2 * (NDISC + j), pl.ds(s, L)]
      m = xv[2 * (NDISC + j) + 1, pl.ds(s, L)]
      u = plsc.bitcast(v * m, jnp.int32)
      u = (u + 0x7FFF + ((u >> 16) & 1)) & ~0xFFFF
      cv = plsc.bitcast(u, jnp.float32)
      for d in range(EDIM):
        a[d] = a[d] + cv * wsp[d][j]
    for d in range(EDIM):
      acc[d, pl.ds(s, L)] = a[d]
    return 0

  lax.fori_loop(0, BPW // L, cont_chunk, 0)

  for f in range(NDISC):
    slot = f % NBUF
    drain(f)
    mrow = 2 * f + 1
    rbase = slot * SLOT

    def emb_chunk(c, _, mrow=mrow, rbase=rbase):
      s = c * L
      m = xv[mrow, pl.ds(s, L)]
      for d in range(EDIM):
        g = rows[pl.ds(rbase + d * BPW + s, L)]
        plsc.addupdate(acc.at[d, pl.ds(s, L)], g * m)
      return 0

    lax.fori_loop(0, BPW // L, emb_chunk, 0)
    nxt = f + NBUF - 1
    if nxt < NDISC:
      fire(nxt)

  pltpu.sync_copy(acc, out_hbm.at[:, pl.ds(base, BPW)])


@functools.partial(
    pl.kernel,
    out_type=jax.ShapeDtypeStruct((EDIM, B), jnp.float32),
    mesh=plsc.VectorSubcoreMesh(core_axis_name="c", subcore_axis_name="s",
                                num_cores=NC, num_subcores=NS),
    compiler_params=pltpu.CompilerParams(needs_layout_passes=False,
                                         use_tc_tiling_on_sc=False),
    scratch_types=[
        pltpu.VMEM((2 * (NDISC + NCONT), BPW), jnp.float32),
        pltpu.VMEM((NCHUNK, CHUNK), jnp.int32),
        pltpu.VMEM((NBUF * SLOT,), jnp.float32),
        pltpu.VMEM((EDIM, BPW), jnp.float32),
        pltpu.VMEM((48, L), jnp.float32),
        pltpu.SemaphoreType.DMA,
        pltpu.SemaphoreType.DMA,
        pltpu.SemaphoreType.DMA,
        pltpu.SemaphoreType.DMA,
    ],
)
def _wide_sc(xt_hbm, tab_hbm, wb_hbm, out_hbm, xv, idxv, rows, acc, wbv,
             sem0, sem1, sem2, sem3):
  _wide_body(xt_hbm, tab_hbm, wb_hbm, out_hbm, xv, idxv, rows, acc, wbv,
             sem0, sem1, sem2, sem3)


def kernel(x, tables, W, b):
  xt = x.T
  tab = tables.transpose(2, 0, 1).reshape(-1)
  wb = jnp.concatenate(
      [W.reshape(-1), b, jnp.zeros((48 - NCONT * EDIM - EDIM,), jnp.float32)])
  wb = jnp.broadcast_to(wb[:, None], (48, L))
  out_t = _wide_sc(xt, tab, wb)
  return out_t.T
